# baseline (speedup 1.0000x reference)
import jax
import jax.numpy as jnp
from jax.experimental import pallas as pl
from jax.experimental.pallas import tpu as pltpu

_B, _C, _H, _W = 1024, 256, 14, 14
_HW = _H * _W


def _mul_body(idx_ref, m_ref, x_ref, o_ref):
    del idx_ref
    m = m_ref[0, 0, :].astype(jnp.float32)
    o_ref[...] = x_ref[...] * m[None, :, None]


def kernel(X, idx, masks):
    n = masks.shape[0]
    x3 = X.reshape(_B, _C, _HW)
    m3 = masks.reshape(n, 1, _C)
    grid_spec = pltpu.PrefetchScalarGridSpec(
        num_scalar_prefetch=1,
        grid=(_B,),
        in_specs=[
            pl.BlockSpec((1, 1, _C), lambda i, idx_ref: (idx_ref[i], 0, 0)),
            pl.BlockSpec((1, _C, _HW), lambda i, idx_ref: (i, 0, 0)),
        ],
        out_specs=pl.BlockSpec((1, _C, _HW), lambda i, idx_ref: (i, 0, 0)),
    )
    out = pl.pallas_call(
        _mul_body,
        grid_spec=grid_spec,
        out_shape=jax.ShapeDtypeStruct((_B, _C, _HW), jnp.float32),
        compiler_params=pltpu.CompilerParams(
            dimension_semantics=("arbitrary",),
        ),
    )(idx.astype(jnp.int32), m3, x3)
    return out.reshape(_B, _C, _H, _W)
